# Initial kernel scaffold; baseline (speedup 1.0000x reference)
import jax
import jax.numpy as jnp
from jax.experimental import pallas as pl


def kernel(meta_real, meta_imag, residual, wq_w, wk_w, wv_w, wo_w, norm_w):
    raise NotImplementedError("write your pallas kernel here")


# speedup vs baseline: 14.1467x; 14.1467x over previous
import functools

import jax
import jax.numpy as jnp
from jax.experimental import pallas as pl
from jax.experimental.pallas import tpu as pltpu

EPS = 1e-5
QEPS = 1e-8
L_BLK = 512
L_BLK_ADD = 1024


def _kq_body(qin_ref, wq_ref, wk_ref, kq_ref, *, scale):
    qin = qin_ref[...]
    g = jnp.clip(jnp.max(jnp.abs(qin), axis=-1, keepdims=True), QEPS, None)
    qa = jnp.round(qin * (127.0 / g)) * (g / 127.0)
    wq = wq_ref[...]
    sq = jnp.mean(jnp.abs(wq)) + QEPS
    wqq = jnp.clip(jnp.round(wq / sq), -1.0, 1.0) * sq
    q_flat = jax.lax.dot_general(qa, wqq, (((1,), (1,)), ((), ())),
                                 preferred_element_type=jnp.float32)
    wk = wk_ref[...]
    sk = jnp.mean(jnp.abs(wk)) + QEPS
    wkq = jnp.clip(jnp.round(wk / sk), -1.0, 1.0) * sk
    kq = jax.lax.dot_general(q_flat, wkq, (((1,), (0,)), ((), ())),
                             preferred_element_type=jnp.float32)
    kq_ref[...] = kq * scale


def _pool_body(kq_ref, nw_ref, r_ref, sx_ref, acc_ref, m_ref, s_ref, *,
               d_model):
    l = pl.program_id(1)
    nl = pl.num_programs(1)

    @pl.when(l == 0)
    def _init():
        m_ref[...] = jnp.full_like(m_ref, -1e30)
        s_ref[...] = jnp.zeros_like(s_ref)
        acc_ref[...] = jnp.zeros_like(acc_ref)

    r = r_ref[0]
    w = nw_ref[...]
    ssq = jnp.sum(r * r, axis=-1, keepdims=True)
    rs = jax.lax.rsqrt(ssq / d_model + EPS)
    u = r * w
    gu = jnp.max(jnp.abs(u), axis=-1, keepdims=True)
    g = jnp.clip(rs * gu, QEPS, None)
    rq = jnp.round(u * (rs * (127.0 / g)))
    c2 = g * (1.0 / 127.0)
    lg = jnp.sum(rq * kq_ref[...], axis=-1, keepdims=True) * c2

    m_old = m_ref[...]
    m_new = jnp.maximum(m_old, jnp.max(lg, axis=0, keepdims=True))
    alpha = jnp.exp(m_old - m_new)
    p = jnp.exp(lg - m_new)
    s_ref[...] = s_ref[...] * alpha + jnp.sum(p, axis=0, keepdims=True)
    pw = p * c2
    acc_ref[...] = acc_ref[...] * alpha + jnp.sum(pw * rq, axis=0,
                                                  keepdims=True)
    m_ref[...] = m_new

    @pl.when(l == nl - 1)
    def _fin():
        sx_ref[...] = (acc_ref[...] / s_ref[...])[:, None, :]


def _tail_body(sx_ref, wv_ref, wo_ref, out_ref):
    sv = jnp.mean(jnp.abs(wv_ref[...])) + QEPS
    wvq = jnp.clip(jnp.round(wv_ref[...] / sv), -1.0, 1.0) * sv
    summary = jax.lax.dot_general(sx_ref[...], wvq, (((1,), (1,)), ((), ())),
                                  preferred_element_type=jnp.float32)
    g = jnp.clip(jnp.max(jnp.abs(summary), axis=-1, keepdims=True), QEPS, None)
    qs = jnp.round(summary * (127.0 / g)) * (g / 127.0)
    so = jnp.mean(jnp.abs(wo_ref[...])) + QEPS
    woq = jnp.clip(jnp.round(wo_ref[...] / so), -1.0, 1.0) * so
    out_ref[...] = jax.lax.dot_general(qs, woq, (((1,), (1,)), ((), ())),
                                       preferred_element_type=jnp.float32)


def _add_body(ov_ref, r_ref, o_ref):
    o_ref[...] = r_ref[...] + ov_ref[...]


def kernel(meta_real, meta_imag, residual, wq_w, wk_w, wv_w, wo_w, norm_w):
    B, L, D = residual.shape
    scale = D ** (-0.5)
    q_input = jnp.stack([meta_real, meta_imag], axis=-1).reshape(1, -1)
    nw = norm_w.reshape(1, D)

    kq = pl.pallas_call(
        functools.partial(_kq_body, scale=scale),
        out_shape=jax.ShapeDtypeStruct((1, D), jnp.float32),
    )(q_input, wq_w, wk_w)

    nl = L // L_BLK
    sx = pl.pallas_call(
        functools.partial(_pool_body, d_model=D),
        grid=(B, nl),
        in_specs=[
            pl.BlockSpec((1, D), lambda b, l: (0, 0)),
            pl.BlockSpec((1, D), lambda b, l: (0, 0)),
            pl.BlockSpec((1, L_BLK, D), lambda b, l: (b, l, 0)),
        ],
        out_specs=pl.BlockSpec((1, 1, D), lambda b, l: (b, 0, 0)),
        out_shape=jax.ShapeDtypeStruct((B, 1, D), jnp.float32),
        scratch_shapes=[
            pltpu.VMEM((1, D), jnp.float32),
            pltpu.VMEM((1, 1), jnp.float32),
            pltpu.VMEM((1, 1), jnp.float32),
        ],
        compiler_params=pltpu.CompilerParams(
            dimension_semantics=("parallel", "arbitrary")),
    )(kq, nw, residual)

    out_vec = pl.pallas_call(
        _tail_body,
        out_shape=jax.ShapeDtypeStruct((B, D), jnp.float32),
    )(sx.reshape(B, D), wv_w, wo_w).reshape(B, 1, D)

    nl2 = L // L_BLK_ADD
    out = pl.pallas_call(
        _add_body,
        grid=(B, nl2),
        in_specs=[
            pl.BlockSpec((1, 1, D), lambda b, l: (b, 0, 0)),
            pl.BlockSpec((1, L_BLK_ADD, D), lambda b, l: (b, l, 0)),
        ],
        out_specs=pl.BlockSpec((1, L_BLK_ADD, D), lambda b, l: (b, l, 0)),
        out_shape=jax.ShapeDtypeStruct((B, L, D), jnp.float32),
        compiler_params=pltpu.CompilerParams(
            dimension_semantics=("parallel", "arbitrary")),
    )(out_vec, residual)
    return out


# speedup vs baseline: 15.7019x; 1.1099x over previous
import functools

import jax
import jax.numpy as jnp
from jax.experimental import pallas as pl
from jax.experimental.pallas import tpu as pltpu

EPS = 1e-5
QEPS = 1e-8
L_BLK = 1024
L_BLK_ADD = 2048


def _kq_body(qin_ref, wq_ref, wk_ref, kq_ref, *, scale):
    qin = qin_ref[...]
    g = jnp.clip(jnp.max(jnp.abs(qin), axis=-1, keepdims=True), QEPS, None)
    qa = jnp.round(qin * (127.0 / g)) * (g / 127.0)
    wq = wq_ref[...]
    sq = jnp.mean(jnp.abs(wq)) + QEPS
    wqq = jnp.clip(jnp.round(wq / sq), -1.0, 1.0) * sq
    q_flat = jax.lax.dot_general(qa, wqq, (((1,), (1,)), ((), ())),
                                 preferred_element_type=jnp.float32)
    wk = wk_ref[...]
    sk = jnp.mean(jnp.abs(wk)) + QEPS
    wkq = jnp.clip(jnp.round(wk / sk), -1.0, 1.0) * sk
    kq = jax.lax.dot_general(q_flat, wkq, (((1,), (0,)), ((), ())),
                             preferred_element_type=jnp.float32)
    kq_ref[...] = kq * scale


def _pool_body(kq_ref, nw_ref, r_ref, sx_ref, acc_ref, m_ref, s_ref, *,
               d_model):
    l = pl.program_id(1)
    nl = pl.num_programs(1)

    @pl.when(l == 0)
    def _init():
        m_ref[...] = jnp.full_like(m_ref, -1e30)
        s_ref[...] = jnp.zeros_like(s_ref)
        acc_ref[...] = jnp.zeros_like(acc_ref)

    r = r_ref[0]
    w = nw_ref[...]
    ssq = jnp.sum(r * r, axis=-1, keepdims=True)
    rs = jax.lax.rsqrt(ssq / d_model + EPS)
    u = r * w
    gu = jnp.max(jnp.abs(u), axis=-1, keepdims=True)
    g = jnp.clip(rs * gu, QEPS, None)
    rq = jnp.round(u * (rs * (127.0 / g)))
    c2 = g * (1.0 / 127.0)
    lg = jnp.sum(rq * kq_ref[...], axis=-1, keepdims=True) * c2

    m_old = m_ref[...]
    m_new = jnp.maximum(m_old, jnp.max(lg, axis=0, keepdims=True))
    alpha = jnp.exp(m_old - m_new)
    p = jnp.exp(lg - m_new)
    s_ref[...] = s_ref[...] * alpha + jnp.sum(p, axis=0, keepdims=True)
    pw = p * c2
    acc_ref[...] = acc_ref[...] * alpha + jnp.sum(pw * rq, axis=0,
                                                  keepdims=True)
    m_ref[...] = m_new

    @pl.when(l == nl - 1)
    def _fin():
        sx_ref[...] = (acc_ref[...] / s_ref[...])[:, None, :]


def _tail_body(sx_ref, wv_ref, wo_ref, out_ref):
    sv = jnp.mean(jnp.abs(wv_ref[...])) + QEPS
    wvq = jnp.clip(jnp.round(wv_ref[...] / sv), -1.0, 1.0) * sv
    summary = jax.lax.dot_general(sx_ref[...], wvq, (((1,), (1,)), ((), ())),
                                  preferred_element_type=jnp.float32)
    g = jnp.clip(jnp.max(jnp.abs(summary), axis=-1, keepdims=True), QEPS, None)
    qs = jnp.round(summary * (127.0 / g)) * (g / 127.0)
    so = jnp.mean(jnp.abs(wo_ref[...])) + QEPS
    woq = jnp.clip(jnp.round(wo_ref[...] / so), -1.0, 1.0) * so
    out_ref[...] = jax.lax.dot_general(qs, woq, (((1,), (1,)), ((), ())),
                                       preferred_element_type=jnp.float32)


def _add_body(ov_ref, r_ref, o_ref):
    o_ref[...] = r_ref[...] + ov_ref[...]


def kernel(meta_real, meta_imag, residual, wq_w, wk_w, wv_w, wo_w, norm_w):
    B, L, D = residual.shape
    scale = D ** (-0.5)
    q_input = jnp.stack([meta_real, meta_imag], axis=-1).reshape(1, -1)
    nw = norm_w.reshape(1, D)

    kq = pl.pallas_call(
        functools.partial(_kq_body, scale=scale),
        out_shape=jax.ShapeDtypeStruct((1, D), jnp.float32),
    )(q_input, wq_w, wk_w)

    nl = L // L_BLK
    sx = pl.pallas_call(
        functools.partial(_pool_body, d_model=D),
        grid=(B, nl),
        in_specs=[
            pl.BlockSpec((1, D), lambda b, l: (0, 0)),
            pl.BlockSpec((1, D), lambda b, l: (0, 0)),
            pl.BlockSpec((1, L_BLK, D), lambda b, l: (b, l, 0)),
        ],
        out_specs=pl.BlockSpec((1, 1, D), lambda b, l: (b, 0, 0)),
        out_shape=jax.ShapeDtypeStruct((B, 1, D), jnp.float32),
        scratch_shapes=[
            pltpu.VMEM((1, D), jnp.float32),
            pltpu.VMEM((1, 1), jnp.float32),
            pltpu.VMEM((1, 1), jnp.float32),
        ],
        compiler_params=pltpu.CompilerParams(
            dimension_semantics=("parallel", "arbitrary"),
            vmem_limit_bytes=56 * 1024 * 1024),
    )(kq, nw, residual)

    out_vec = pl.pallas_call(
        _tail_body,
        out_shape=jax.ShapeDtypeStruct((B, D), jnp.float32),
    )(sx.reshape(B, D), wv_w, wo_w).reshape(B, 1, D)

    nl2 = L // L_BLK_ADD
    out = pl.pallas_call(
        _add_body,
        grid=(B, nl2),
        in_specs=[
            pl.BlockSpec((1, 1, D), lambda b, l: (b, 0, 0)),
            pl.BlockSpec((1, L_BLK_ADD, D), lambda b, l: (b, l, 0)),
        ],
        out_specs=pl.BlockSpec((1, L_BLK_ADD, D), lambda b, l: (b, l, 0)),
        out_shape=jax.ShapeDtypeStruct((B, L, D), jnp.float32),
        compiler_params=pltpu.CompilerParams(
            dimension_semantics=("parallel", "arbitrary"),
            vmem_limit_bytes=56 * 1024 * 1024),
    )(out_vec, residual)
    return out


# speedup vs baseline: 16.3267x; 1.0398x over previous
import functools

import jax
import jax.numpy as jnp
from jax.experimental import pallas as pl
from jax.experimental.pallas import tpu as pltpu

EPS = 1e-5
QEPS = 1e-8
L_BLK = 512


def _prep_body(qin_ref, wq_ref, wk_ref, wv_ref, wo_ref,
               kq_ref, wvq_ref, woq_ref, *, scale):
    qin = qin_ref[...]
    g = jnp.clip(jnp.max(jnp.abs(qin), axis=-1, keepdims=True), QEPS, None)
    qa = jnp.round(qin * (127.0 / g)) * (g / 127.0)
    wq = wq_ref[...]
    sq = jnp.mean(jnp.abs(wq)) + QEPS
    wqq = jnp.clip(jnp.round(wq / sq), -1.0, 1.0) * sq
    q_flat = jax.lax.dot_general(qa, wqq, (((1,), (1,)), ((), ())),
                                 preferred_element_type=jnp.float32)
    wk = wk_ref[...]
    sk = jnp.mean(jnp.abs(wk)) + QEPS
    wkq = jnp.clip(jnp.round(wk / sk), -1.0, 1.0) * sk
    kq = jax.lax.dot_general(q_flat, wkq, (((1,), (0,)), ((), ())),
                             preferred_element_type=jnp.float32)
    kq_ref[...] = kq * scale
    wv = wv_ref[...]
    sv = jnp.mean(jnp.abs(wv)) + QEPS
    wvq_ref[...] = jnp.clip(jnp.round(wv / sv), -1.0, 1.0) * sv
    wo = wo_ref[...]
    so = jnp.mean(jnp.abs(wo)) + QEPS
    woq_ref[...] = jnp.clip(jnp.round(wo / so), -1.0, 1.0) * so


def _mega_body(kq_ref, nw_ref, wvq_ref, woq_ref, r_ref, o_ref,
               resbuf_ref, acc_ref, m_ref, s_ref, corr_ref, *,
               d_model, nl):
    p = pl.program_id(1)
    l = pl.program_id(2)
    off = pl.multiple_of(l * L_BLK, L_BLK)

    @pl.when(p == 0)
    def _pool():
        @pl.when(l == 0)
        def _init():
            m_ref[...] = jnp.full_like(m_ref, -1e30)
            s_ref[...] = jnp.zeros_like(s_ref)
            acc_ref[...] = jnp.zeros_like(acc_ref)

        r = r_ref[0]
        resbuf_ref[pl.ds(off, L_BLK), :] = r
        w = nw_ref[...]
        ssq = jnp.sum(r * r, axis=-1, keepdims=True)
        rs = jax.lax.rsqrt(ssq / d_model + EPS)
        u = r * w
        gu = jnp.max(jnp.abs(u), axis=-1, keepdims=True)
        g = jnp.clip(rs * gu, QEPS, None)
        rq = jnp.round(u * (rs * (127.0 / g)))
        c2 = g * (1.0 / 127.0)
        lg = jnp.sum(rq * kq_ref[...], axis=-1, keepdims=True) * c2

        m_old = m_ref[...]
        m_new = jnp.maximum(m_old, jnp.max(lg, axis=0, keepdims=True))
        alpha = jnp.exp(m_old - m_new)
        pr = jnp.exp(lg - m_new)
        s_ref[...] = s_ref[...] * alpha + jnp.sum(pr, axis=0, keepdims=True)
        pw = pr * c2
        acc_ref[...] = acc_ref[...] * alpha + jnp.sum(pw * rq, axis=0,
                                                      keepdims=True)
        m_ref[...] = m_new

        @pl.when(l == nl - 1)
        def _tail():
            sx = acc_ref[...] / s_ref[...]
            summary = jax.lax.dot_general(
                sx, wvq_ref[...], (((1,), (1,)), ((), ())),
                preferred_element_type=jnp.float32)
            gs = jnp.clip(jnp.max(jnp.abs(summary), axis=-1, keepdims=True),
                          QEPS, None)
            qs = jnp.round(summary * (127.0 / gs)) * (gs / 127.0)
            corr_ref[...] = jax.lax.dot_general(
                qs, woq_ref[...], (((1,), (1,)), ((), ())),
                preferred_element_type=jnp.float32)

    @pl.when(p == 1)
    def _add():
        o_ref[...] = (resbuf_ref[pl.ds(off, L_BLK), :] + corr_ref[...])[None]


def kernel(meta_real, meta_imag, residual, wq_w, wk_w, wv_w, wo_w, norm_w):
    B, L, D = residual.shape
    scale = D ** (-0.5)
    q_input = jnp.stack([meta_real, meta_imag], axis=-1).reshape(1, -1)
    nw = norm_w.reshape(1, D)

    kq, wvq, woq = pl.pallas_call(
        functools.partial(_prep_body, scale=scale),
        out_shape=(
            jax.ShapeDtypeStruct((1, D), jnp.float32),
            jax.ShapeDtypeStruct((D, D), jnp.float32),
            jax.ShapeDtypeStruct((D, D), jnp.float32),
        ),
    )(q_input, wq_w, wk_w, wv_w, wo_w)

    nl = L // L_BLK
    out = pl.pallas_call(
        functools.partial(_mega_body, d_model=D, nl=nl),
        grid=(B, 2, nl),
        in_specs=[
            pl.BlockSpec((1, D), lambda b, p, l: (0, 0)),
            pl.BlockSpec((1, D), lambda b, p, l: (0, 0)),
            pl.BlockSpec((D, D), lambda b, p, l: (0, 0)),
            pl.BlockSpec((D, D), lambda b, p, l: (0, 0)),
            pl.BlockSpec((1, L_BLK, D),
                         lambda b, p, l: (b, jnp.where(p == 0, l, 0), 0)),
        ],
        out_specs=pl.BlockSpec((1, L_BLK, D),
                               lambda b, p, l: (b, jnp.where(p == 0, 0, l), 0)),
        out_shape=jax.ShapeDtypeStruct((B, L, D), jnp.float32),
        scratch_shapes=[
            pltpu.VMEM((L, D), jnp.float32),
            pltpu.VMEM((1, D), jnp.float32),
            pltpu.VMEM((1, 1), jnp.float32),
            pltpu.VMEM((1, 1), jnp.float32),
            pltpu.VMEM((1, D), jnp.float32),
        ],
        compiler_params=pltpu.CompilerParams(
            dimension_semantics=("parallel", "arbitrary", "arbitrary"),
            vmem_limit_bytes=56 * 1024 * 1024),
    )(kq, nw, wvq, woq, residual)
    return out


# speedup vs baseline: 16.5354x; 1.0128x over previous
import functools

import jax
import jax.numpy as jnp
from jax.experimental import pallas as pl
from jax.experimental.pallas import tpu as pltpu

EPS = 1e-5
QEPS = 1e-8
L_BLK = 512


def _prep_body(qin_ref, wq_ref, wk_ref, wv_ref, wo_ref,
               kq_ref, wvq_ref, woq_ref, *, scale):
    qin = qin_ref[...]
    g = jnp.clip(jnp.max(jnp.abs(qin), axis=-1, keepdims=True), QEPS, None)
    qa = jnp.round(qin * (127.0 / g)) * (g / 127.0)
    wq = wq_ref[...]
    sq = jnp.mean(jnp.abs(wq)) + QEPS
    wqq = jnp.clip(jnp.round(wq / sq), -1.0, 1.0) * sq
    q_flat = jax.lax.dot_general(qa, wqq, (((1,), (1,)), ((), ())),
                                 preferred_element_type=jnp.float32)
    wk = wk_ref[...]
    sk = jnp.mean(jnp.abs(wk)) + QEPS
    wkq = jnp.clip(jnp.round(wk / sk), -1.0, 1.0) * sk
    kq = jax.lax.dot_general(q_flat, wkq, (((1,), (0,)), ((), ())),
                             preferred_element_type=jnp.float32)
    kq_ref[...] = kq * scale
    wv = wv_ref[...]
    sv = jnp.mean(jnp.abs(wv)) + QEPS
    wvq_ref[...] = jnp.clip(jnp.round(wv / sv), -1.0, 1.0) * sv
    wo = wo_ref[...]
    so = jnp.mean(jnp.abs(wo)) + QEPS
    woq_ref[...] = jnp.clip(jnp.round(wo / so), -1.0, 1.0) * so


def _mega_body(kq_ref, nw_ref, wvq_ref, woq_ref, r_ref, o_ref,
               resbuf_ref, acc_ref, m_ref, s_ref, corr_ref, *,
               d_model, nl):
    p = pl.program_id(1)
    l = pl.program_id(2)
    off = pl.multiple_of(l * L_BLK, L_BLK)

    @pl.when(p == 0)
    def _pool():
        @pl.when(l == 0)
        def _init():
            m_ref[...] = jnp.full_like(m_ref, -1e30)
            s_ref[...] = jnp.zeros_like(s_ref)
            acc_ref[...] = jnp.zeros_like(acc_ref)

        r = r_ref[0]
        resbuf_ref[pl.ds(off, L_BLK), :] = r
        w = nw_ref[...]
        ssq = jnp.sum(r * r, axis=-1, keepdims=True)
        rs = jax.lax.rsqrt(ssq / d_model + EPS)
        u = r * w
        gu = jnp.max(jnp.abs(u), axis=-1, keepdims=True)
        g = jnp.clip(rs * gu, QEPS, None)
        rq = jnp.round(u * (rs * (127.0 / g)))
        c2 = g * (1.0 / 127.0)
        lg = jnp.sum(rq * kq_ref[...], axis=-1, keepdims=True) * c2
        rq_bf = rq.astype(jnp.bfloat16)

        m_old = m_ref[...]
        m_new = jnp.maximum(m_old, jnp.max(lg, axis=0, keepdims=True))
        alpha = jnp.exp(m_old - m_new)
        pr = jnp.exp(lg - m_new)
        s_ref[...] = s_ref[...] * alpha + jnp.sum(pr, axis=0, keepdims=True)
        pw = (pr * c2).astype(jnp.bfloat16)
        acc_ref[...] = acc_ref[...] * alpha + jax.lax.dot_general(
            pw, rq_bf, (((0,), (0,)), ((), ())),
            preferred_element_type=jnp.float32)
        m_ref[...] = m_new

        @pl.when(l == nl - 1)
        def _tail():
            sx = acc_ref[...] / s_ref[...]
            summary = jax.lax.dot_general(
                sx, wvq_ref[...], (((1,), (1,)), ((), ())),
                preferred_element_type=jnp.float32)
            gs = jnp.clip(jnp.max(jnp.abs(summary), axis=-1, keepdims=True),
                          QEPS, None)
            qs = jnp.round(summary * (127.0 / gs)) * (gs / 127.0)
            corr_ref[...] = jax.lax.dot_general(
                qs, woq_ref[...], (((1,), (1,)), ((), ())),
                preferred_element_type=jnp.float32)

    @pl.when(p == 1)
    def _add():
        o_ref[...] = (resbuf_ref[pl.ds(off, L_BLK), :] + corr_ref[...])[None]


def kernel(meta_real, meta_imag, residual, wq_w, wk_w, wv_w, wo_w, norm_w):
    B, L, D = residual.shape
    scale = D ** (-0.5)
    q_input = jnp.stack([meta_real, meta_imag], axis=-1).reshape(1, -1)
    nw = norm_w.reshape(1, D)

    kq, wvq, woq = pl.pallas_call(
        functools.partial(_prep_body, scale=scale),
        out_shape=(
            jax.ShapeDtypeStruct((1, D), jnp.float32),
            jax.ShapeDtypeStruct((D, D), jnp.float32),
            jax.ShapeDtypeStruct((D, D), jnp.float32),
        ),
    )(q_input, wq_w, wk_w, wv_w, wo_w)

    nl = L // L_BLK
    out = pl.pallas_call(
        functools.partial(_mega_body, d_model=D, nl=nl),
        grid=(B, 2, nl),
        in_specs=[
            pl.BlockSpec((1, D), lambda b, p, l: (0, 0)),
            pl.BlockSpec((1, D), lambda b, p, l: (0, 0)),
            pl.BlockSpec((D, D), lambda b, p, l: (0, 0)),
            pl.BlockSpec((D, D), lambda b, p, l: (0, 0)),
            pl.BlockSpec((1, L_BLK, D),
                         lambda b, p, l: (b, jnp.where(p == 0, l, 0), 0)),
        ],
        out_specs=pl.BlockSpec((1, L_BLK, D),
                               lambda b, p, l: (b, jnp.where(p == 0, 0, l), 0)),
        out_shape=jax.ShapeDtypeStruct((B, L, D), jnp.float32),
        scratch_shapes=[
            pltpu.VMEM((L, D), jnp.float32),
            pltpu.VMEM((1, D), jnp.float32),
            pltpu.VMEM((1, 1), jnp.float32),
            pltpu.VMEM((1, 1), jnp.float32),
            pltpu.VMEM((1, D), jnp.float32),
        ],
        compiler_params=pltpu.CompilerParams(
            dimension_semantics=("parallel", "arbitrary", "arbitrary"),
            vmem_limit_bytes=56 * 1024 * 1024),
    )(kq, nw, wvq, woq, residual)
    return out
